# baseline (speedup 1.0000x reference)
import functools

import jax
import jax.numpy as jnp
from jax import lax
from jax.experimental import pallas as pl
from jax.experimental.pallas import tpu as pltpu
from jax.experimental.pallas import tpu_sc as plsc

_NC = 2
_NS = 16
_NW = _NC * _NS
_K = 64
_NBUF = 4


@functools.lru_cache(maxsize=None)
def _make_agg(N, E, D):
    EW = -(-E // _NW)
    n_chunks = ((EW + _K - 1) // _K + _NBUF - 1) // _NBUF * _NBUF
    NP = n_chunks * _K
    NPAD = (N + _NS * 8 - 1) // (_NS * 8) * (_NS * 8)
    zstripe = NPAD // _NS
    stripe = (N // _NS) // 8 * 8
    tail = N - stripe * _NS

    mesh = plsc.VectorSubcoreMesh(core_axis_name="c", subcore_axis_name="s")

    @functools.partial(
        pl.kernel,
        out_type=[
            jax.ShapeDtypeStruct((_NC * N, D), jnp.float32),
            jax.ShapeDtypeStruct((_NW * N,), jnp.float32),
        ],
        mesh=mesh,
        scratch_types=[
            pltpu.VMEM((_NBUF, _K), jnp.int32),
            pltpu.VMEM((_NBUF, _K), jnp.int32),
            pltpu.VMEM((_NBUF, _K, D), jnp.float32),
            pltpu.VMEM((NPAD,), jnp.float32),
            pltpu.VMEM_SHARED((NPAD, D), jnp.float32),
            [pltpu.SemaphoreType.DMA] * _NBUF,
            [pltpu.SemaphoreType.DMA] * _NBUF,
            pltpu.SemaphoreType.DMA,
        ],
        compiler_params=pltpu.CompilerParams(needs_layout_passes=False),
    )
    def agg(src_h, dst_h, x_h, z_h, acc_out, deg_out,
            src_v, dst_v, rows_v, deg_v, acc_sh, gsems, isems, ssem):
        i32 = jnp.int32
        cid = lax.axis_index("c")
        sid = lax.axis_index("s")
        wid = sid * i32(_NC) + cid

        zoff = sid * i32(zstripe)
        pltpu.sync_copy(z_h.at[pl.ds(zoff, zstripe)],
                        acc_sh.at[pl.ds(zoff, zstripe)])

        def zbody(i, carry):
            deg_v[pl.ds(i * i32(16), 16)] = jnp.zeros((16,), jnp.float32)
            return carry
        lax.fori_loop(i32(0), i32(NPAD // 16), zbody, i32(0))

        plsc.subcore_barrier()

        ones = jnp.ones((16,), jnp.float32)

        def iload(c, p):
            for ref, hbm in ((src_v, src_h), (dst_v, dst_h)):
                pltpu.make_async_copy(hbm.at[wid, c], ref.at[i32(p)],
                                      isems[p]).start()

        def iload_wait(p):
            for ref, hbm in ((src_v, src_h), (dst_v, dst_h)):
                pltpu.make_async_copy(hbm.at[wid, i32(0)], ref.at[i32(p)],
                                      isems[p]).wait()

        def gather(p):
            pltpu.make_async_copy(x_h.at[src_v.at[i32(p)]],
                                  rows_v.at[i32(p)], gsems[p]).start()

        def gather_wait(p):
            pltpu.make_async_copy(x_h.at[src_v.at[i32(0)]],
                                  rows_v.at[i32(p)], gsems[p]).wait()

        iload(i32(0), 0)
        iload(i32(1), 1)
        iload(i32(2), 2)
        iload_wait(0)
        gather(0)
        iload_wait(1)
        gather(1)

        def quarter(c, q):
            @pl.when(c < i32(n_chunks - 3))
            def _stage_idx():
                iload(c + i32(3), (q + 3) % _NBUF)
            gather_wait(q)
            scat = pltpu.make_async_copy(rows_v.at[i32(q)],
                                         acc_sh.at[dst_v.at[i32(q)]], ssem)
            scat.start(add=True)
            for j in range(_K // 16):
                dvec = dst_v[i32(q), pl.ds(i32(j * 16), 16)]
                plsc.addupdate_scatter(deg_v, [dvec], ones)
            scat.wait()

            @pl.when(c < i32(n_chunks - 2))
            def _next_gather():
                iload_wait((q + 2) % _NBUF)
                gather((q + 2) % _NBUF)

        def step(s, carry):
            for q in range(_NBUF):
                quarter(s * i32(_NBUF) + i32(q), q)
            return carry
        lax.fori_loop(i32(0), i32(n_chunks // _NBUF), step, i32(0))

        plsc.subcore_barrier()

        soff = sid * i32(stripe)
        pltpu.sync_copy(acc_sh.at[pl.ds(soff, stripe)],
                        acc_out.at[pl.ds(cid * i32(N) + soff, stripe)])
        if tail:
            @pl.when(sid == _NS - 1)
            def _out_tail():
                pltpu.sync_copy(
                    acc_sh.at[pl.ds(_NS * stripe, tail)],
                    acc_out.at[pl.ds(cid * i32(N) + i32(_NS * stripe), tail)])
        pltpu.sync_copy(deg_v.at[pl.ds(i32(0), N)],
                        deg_out.at[pl.ds(wid * i32(N), N)])

    return agg, NP, NPAD


def _dense_body(acc_ref, deg_ref, x_ref, wl_ref, bl_ref, wr_ref, wo_ref,
                bo_ref, out_ref):
    agg_sum = acc_ref[0] + acc_ref[1]
    deg = jnp.sum(deg_ref[...], axis=1, keepdims=True)
    agg = agg_sum * (1.0 / jnp.maximum(deg, 1.0))
    h = jnp.dot(agg, wl_ref[...], preferred_element_type=jnp.float32,
                precision=lax.Precision.HIGHEST)
    h = h + jnp.dot(x_ref[...], wr_ref[...], preferred_element_type=jnp.float32,
                    precision=lax.Precision.HIGHEST)
    h = h + bl_ref[...]
    h = jnp.maximum(h, 0.0)
    z = jnp.dot(h, wo_ref[...], preferred_element_type=jnp.float32,
                precision=lax.Precision.HIGHEST) + bo_ref[...]
    out_ref[...] = jax.nn.sigmoid(z)


@functools.lru_cache(maxsize=None)
def _make_dense(N, D, C, BN):
    grid = (N // BN,)

    def _z(i):
        return jnp.zeros_like(i)

    return pl.pallas_call(
        _dense_body,
        grid=grid,
        in_specs=[
            pl.BlockSpec((_NC, BN, D), lambda i: (_z(i), i, _z(i))),
            pl.BlockSpec((BN, _NW), lambda i: (i, _z(i))),
            pl.BlockSpec((BN, D), lambda i: (i, _z(i))),
            pl.BlockSpec((D, D), lambda i: (_z(i), _z(i))),
            pl.BlockSpec((1, D), lambda i: (_z(i), _z(i))),
            pl.BlockSpec((D, D), lambda i: (_z(i), _z(i))),
            pl.BlockSpec((D, C), lambda i: (_z(i), _z(i))),
            pl.BlockSpec((1, C), lambda i: (_z(i), _z(i))),
        ],
        out_specs=pl.BlockSpec((BN, C), lambda i: (i, _z(i))),
        out_shape=jax.ShapeDtypeStruct((N, C), jnp.float32),
    )


def kernel(x, edge_index, W_l, b_l, W_r, W_out, b_out):
    N, D = x.shape
    E = edge_index.shape[1]
    C = W_out.shape[0]

    agg_fn, NP, NPAD = _make_agg(N, E, D)

    ei = edge_index.astype(jnp.int32)
    EW = E // _NW
    src = ei[0].reshape(_NW, EW)
    dst = ei[1].reshape(_NW, EW)
    pad = ((0, 0), (0, NP - EW))
    src = jnp.pad(src, pad).reshape(_NW, NP // _K, _K)
    dst = jnp.pad(dst, pad, constant_values=N).reshape(_NW, NP // _K, _K)
    zeros = jnp.zeros((NPAD, D), jnp.float32)

    acc_flat, deg_flat = agg_fn(src, dst, x, zeros)
    acc = acc_flat.reshape(_NC, N, D)
    deg = deg_flat.reshape(_NW, N).T

    out = _make_dense(N, D, C, 2000)(
        acc, deg, x,
        W_l.T, b_l.reshape(1, D), W_r.T, W_out.T, b_out.reshape(1, C))
    return out


# speedup vs baseline: 1.1676x; 1.1676x over previous
import functools

import jax
import jax.numpy as jnp
from jax import lax
from jax.experimental import pallas as pl
from jax.experimental.pallas import tpu as pltpu
from jax.experimental.pallas import tpu_sc as plsc

_NC = 2
_NS = 16
_NW = _NC * _NS
_K = 96
_NBUF = 2


@functools.lru_cache(maxsize=None)
def _make_agg(N, E, D):
    EW = -(-E // _NW)
    n_chunks = ((EW + _K - 1) // _K + 1) // 2 * 2
    NP = n_chunks * _K
    NPAD = (N + _NS * 8 - 1) // (_NS * 8) * (_NS * 8)
    zstripe = NPAD // _NS
    stripe = (N // _NS) // 8 * 8
    tail = N - stripe * _NS

    mesh = plsc.VectorSubcoreMesh(core_axis_name="c", subcore_axis_name="s")

    @functools.partial(
        pl.kernel,
        out_type=[
            jax.ShapeDtypeStruct((_NC * N, D), jnp.float32),
            jax.ShapeDtypeStruct((_NW * N,), jnp.float32),
        ],
        mesh=mesh,
        scratch_types=[
            pltpu.VMEM((2, _K), jnp.int32),
            pltpu.VMEM((2, _K), jnp.int32),
            pltpu.VMEM((2, _K, D), jnp.float32),
            pltpu.VMEM((NPAD,), jnp.float32),
            pltpu.VMEM_SHARED((NPAD, D), jnp.float32),
            [pltpu.SemaphoreType.DMA] * 2,
            [pltpu.SemaphoreType.DMA] * 2,
            pltpu.SemaphoreType.DMA,
        ],
        compiler_params=pltpu.CompilerParams(needs_layout_passes=False),
    )
    def agg(src_h, dst_h, x_h, z_h, acc_out, deg_out,
            src_v, dst_v, rows_v, deg_v, acc_sh, gsems, isems, ssem):
        i32 = jnp.int32
        cid = lax.axis_index("c")
        sid = lax.axis_index("s")
        wid = sid * i32(_NC) + cid

        zoff = sid * i32(zstripe)
        pltpu.sync_copy(z_h.at[pl.ds(zoff, zstripe)],
                        acc_sh.at[pl.ds(zoff, zstripe)])

        def zbody(i, carry):
            deg_v[pl.ds(i * i32(16), 16)] = jnp.zeros((16,), jnp.float32)
            return carry
        lax.fori_loop(i32(0), i32(NPAD // 16), zbody, i32(0))

        plsc.subcore_barrier()

        ones = jnp.ones((16,), jnp.float32)

        def iload(c, p):
            for ref, hbm in ((src_v, src_h), (dst_v, dst_h)):
                pltpu.make_async_copy(hbm.at[wid, c], ref.at[i32(p)],
                                      isems[p]).start()

        def iload_wait(p):
            for ref, hbm in ((src_v, src_h), (dst_v, dst_h)):
                pltpu.make_async_copy(hbm.at[wid, i32(0)], ref.at[i32(p)],
                                      isems[p]).wait()

        def gather(p):
            pltpu.make_async_copy(x_h.at[src_v.at[i32(p)]],
                                  rows_v.at[i32(p)], gsems[p]).start()

        def gather_wait(p):
            pltpu.make_async_copy(x_h.at[src_v.at[i32(0)]],
                                  rows_v.at[i32(p)], gsems[p]).wait()

        iload(i32(0), 0)
        iload_wait(0)
        gather(0)
        iload(i32(1), 1)

        def half(c, p):
            @pl.when(c < i32(n_chunks - 1))
            def _next_gather():
                iload_wait(1 - p)
                gather(1 - p)
            gather_wait(p)
            scat = pltpu.make_async_copy(rows_v.at[i32(p)],
                                         acc_sh.at[dst_v.at[i32(p)]], ssem)
            scat.start(add=True)
            for j in range(_K // 16):
                dvec = dst_v[i32(p), pl.ds(i32(j * 16), 16)]
                plsc.addupdate_scatter(deg_v, [dvec], ones)
            scat.wait()

            @pl.when(c < i32(n_chunks - 2))
            def _next_iload():
                iload(c + i32(2), p)

        def step(s, carry):
            half(s * i32(2), 0)
            half(s * i32(2) + i32(1), 1)
            return carry
        lax.fori_loop(i32(0), i32(n_chunks // 2), step, i32(0))

        plsc.subcore_barrier()

        soff = sid * i32(stripe)
        pltpu.sync_copy(acc_sh.at[pl.ds(soff, stripe)],
                        acc_out.at[pl.ds(cid * i32(N) + soff, stripe)])
        if tail:
            @pl.when(sid == _NS - 1)
            def _out_tail():
                pltpu.sync_copy(
                    acc_sh.at[pl.ds(_NS * stripe, tail)],
                    acc_out.at[pl.ds(cid * i32(N) + i32(_NS * stripe), tail)])
        pltpu.sync_copy(deg_v.at[pl.ds(i32(0), N)],
                        deg_out.at[pl.ds(wid * i32(N), N)])

    return agg, NP, NPAD


def _dense_body(acc_ref, deg_ref, x_ref, wl_ref, bl_ref, wr_ref, wo_ref,
                bo_ref, out_ref):
    agg_sum = acc_ref[0] + acc_ref[1]
    deg = jnp.sum(deg_ref[...], axis=1, keepdims=True)
    agg = agg_sum * (1.0 / jnp.maximum(deg, 1.0))
    h = jnp.dot(agg, wl_ref[...], preferred_element_type=jnp.float32,
                precision=lax.Precision.HIGHEST)
    h = h + jnp.dot(x_ref[...], wr_ref[...], preferred_element_type=jnp.float32,
                    precision=lax.Precision.HIGHEST)
    h = h + bl_ref[...]
    h = jnp.maximum(h, 0.0)
    z = jnp.dot(h, wo_ref[...], preferred_element_type=jnp.float32,
                precision=lax.Precision.HIGHEST) + bo_ref[...]
    out_ref[...] = jax.nn.sigmoid(z)


@functools.lru_cache(maxsize=None)
def _make_dense(N, D, C, BN):
    grid = (N // BN,)

    def _z(i):
        return jnp.zeros_like(i)

    return pl.pallas_call(
        _dense_body,
        grid=grid,
        in_specs=[
            pl.BlockSpec((_NC, BN, D), lambda i: (_z(i), i, _z(i))),
            pl.BlockSpec((BN, _NW), lambda i: (i, _z(i))),
            pl.BlockSpec((BN, D), lambda i: (i, _z(i))),
            pl.BlockSpec((D, D), lambda i: (_z(i), _z(i))),
            pl.BlockSpec((1, D), lambda i: (_z(i), _z(i))),
            pl.BlockSpec((D, D), lambda i: (_z(i), _z(i))),
            pl.BlockSpec((D, C), lambda i: (_z(i), _z(i))),
            pl.BlockSpec((1, C), lambda i: (_z(i), _z(i))),
        ],
        out_specs=pl.BlockSpec((BN, C), lambda i: (i, _z(i))),
        out_shape=jax.ShapeDtypeStruct((N, C), jnp.float32),
    )


def kernel(x, edge_index, W_l, b_l, W_r, W_out, b_out):
    N, D = x.shape
    E = edge_index.shape[1]
    C = W_out.shape[0]

    agg_fn, NP, NPAD = _make_agg(N, E, D)

    ei = edge_index.astype(jnp.int32)
    EW = E // _NW
    src = ei[0].reshape(_NW, EW)
    dst = ei[1].reshape(_NW, EW)
    pad = ((0, 0), (0, NP - EW))
    src = jnp.pad(src, pad).reshape(_NW, NP // _K, _K)
    dst = jnp.pad(dst, pad, constant_values=N).reshape(_NW, NP // _K, _K)
    zeros = jnp.zeros((NPAD, D), jnp.float32)

    acc_flat, deg_flat = agg_fn(src, dst, x, zeros)
    acc = acc_flat.reshape(_NC, N, D)
    deg = deg_flat.reshape(_NW, N).T

    out = _make_dense(N, D, C, 2000)(
        acc, deg, x,
        W_l.T, b_l.reshape(1, D), W_r.T, W_out.T, b_out.reshape(1, C))
    return out


# speedup vs baseline: 1.5803x; 1.3534x over previous
import functools

import jax
import jax.numpy as jnp
from jax import lax
from jax.experimental import pallas as pl
from jax.experimental.pallas import tpu as pltpu
from jax.experimental.pallas import tpu_sc as plsc

_NC = 2
_NS = 16
_NW = _NC * _NS
_K = 80
_NBUF = 2


@functools.lru_cache(maxsize=None)
def _make_agg(N, E, D):
    EW = -(-E // _NW)
    n_chunks = ((EW + _K - 1) // _K + 1) // 2 * 2
    NP = n_chunks * _K
    NPAD = (N + _NS * 8 - 1) // (_NS * 8) * (_NS * 8)
    zstripe = NPAD // _NS
    stripe = (N // _NS) // 8 * 8
    tail = N - stripe * _NS

    mesh = plsc.VectorSubcoreMesh(core_axis_name="c", subcore_axis_name="s")

    @functools.partial(
        pl.kernel,
        out_type=[
            jax.ShapeDtypeStruct((_NC * N, D), jnp.float32),
            jax.ShapeDtypeStruct((_NW * N,), jnp.float32),
        ],
        mesh=mesh,
        scratch_types=[
            pltpu.VMEM((2, _K), jnp.int32),
            pltpu.VMEM((2, _K), jnp.int32),
            pltpu.VMEM((2, _K, D), jnp.float32),
            pltpu.VMEM((NPAD,), jnp.float32),
            pltpu.VMEM_SHARED((NPAD, D), jnp.float32),
            [pltpu.SemaphoreType.DMA] * 2,
            [pltpu.SemaphoreType.DMA] * 2,
            pltpu.SemaphoreType.DMA,
        ],
        compiler_params=pltpu.CompilerParams(needs_layout_passes=False),
    )
    def agg(src_h, dst_h, x_h, z_h, acc_out, deg_out,
            src_v, dst_v, rows_v, deg_v, acc_sh, gsems, isems, ssem):
        i32 = jnp.int32
        cid = lax.axis_index("c")
        sid = lax.axis_index("s")
        wid = sid * i32(_NC) + cid

        zoff = sid * i32(zstripe)
        pltpu.sync_copy(z_h.at[pl.ds(zoff, zstripe)],
                        acc_sh.at[pl.ds(zoff, zstripe)])

        def zbody(i, carry):
            deg_v[pl.ds(i * i32(16), 16)] = jnp.zeros((16,), jnp.float32)
            return carry
        lax.fori_loop(i32(0), i32(NPAD // 16), zbody, i32(0))

        plsc.subcore_barrier()

        ones = jnp.ones((16,), jnp.float32)

        def iload(c, p):
            for ref, hbm in ((src_v, src_h), (dst_v, dst_h)):
                pltpu.make_async_copy(hbm.at[wid, c], ref.at[i32(p)],
                                      isems[p]).start()

        def iload_wait(p):
            for ref, hbm in ((src_v, src_h), (dst_v, dst_h)):
                pltpu.make_async_copy(hbm.at[wid, i32(0)], ref.at[i32(p)],
                                      isems[p]).wait()

        def gather(p):
            pltpu.make_async_copy(x_h.at[src_v.at[i32(p)]],
                                  rows_v.at[i32(p)], gsems[p]).start()

        def gather_wait(p):
            pltpu.make_async_copy(x_h.at[src_v.at[i32(0)]],
                                  rows_v.at[i32(p)], gsems[p]).wait()

        iload(i32(0), 0)
        iload_wait(0)
        gather(0)
        iload(i32(1), 1)

        def half(c, p):
            @pl.when(c < i32(n_chunks - 1))
            def _next_gather():
                iload_wait(1 - p)
                gather(1 - p)
            gather_wait(p)
            scat = pltpu.make_async_copy(rows_v.at[i32(p)],
                                         acc_sh.at[dst_v.at[i32(p)]], ssem)
            scat.start(add=True)
            for j in range(_K // 16):
                dvec = dst_v[i32(p), pl.ds(i32(j * 16), 16)]
                plsc.addupdate_scatter(deg_v, [dvec], ones)
            scat.wait()

            @pl.when(c < i32(n_chunks - 2))
            def _next_iload():
                iload(c + i32(2), p)

        def step(s, carry):
            half(s * i32(2), 0)
            half(s * i32(2) + i32(1), 1)
            return carry
        lax.fori_loop(i32(0), i32(n_chunks // 2), step, i32(0))

        plsc.subcore_barrier()

        soff = sid * i32(stripe)
        pltpu.sync_copy(acc_sh.at[pl.ds(soff, stripe)],
                        acc_out.at[pl.ds(cid * i32(N) + soff, stripe)])
        if tail:
            @pl.when(sid == _NS - 1)
            def _out_tail():
                pltpu.sync_copy(
                    acc_sh.at[pl.ds(_NS * stripe, tail)],
                    acc_out.at[pl.ds(cid * i32(N) + i32(_NS * stripe), tail)])
        pltpu.sync_copy(deg_v.at[pl.ds(i32(0), N)],
                        deg_out.at[pl.ds(wid * i32(N), N)])

    return agg, NP, NPAD


def _dense1_body(x_ref, wr_ref, bl_ref, hr_ref):
    hr_ref[...] = jnp.dot(
        x_ref[...], wr_ref[...], preferred_element_type=jnp.float32,
        precision=lax.Precision.HIGHEST) + bl_ref[...]


@functools.lru_cache(maxsize=None)
def _make_dense1(N, D, BN):
    def _z(i):
        return jnp.zeros_like(i)

    return pl.pallas_call(
        _dense1_body,
        grid=(N // BN,),
        in_specs=[
            pl.BlockSpec((BN, D), lambda i: (i, _z(i))),
            pl.BlockSpec((D, D), lambda i: (_z(i), _z(i))),
            pl.BlockSpec((1, D), lambda i: (_z(i), _z(i))),
        ],
        out_specs=pl.BlockSpec((BN, D), lambda i: (i, _z(i))),
        out_shape=jax.ShapeDtypeStruct((N, D), jnp.float32),
    )


def _dense_body(acc_ref, deg_ref, hr_ref, wl_ref, wo_ref, bo_ref, out_ref):
    agg_sum = acc_ref[0] + acc_ref[1]
    deg = jnp.sum(deg_ref[...], axis=1, keepdims=True)
    agg = agg_sum * (1.0 / jnp.maximum(deg, 1.0))
    h = jnp.dot(agg, wl_ref[...], preferred_element_type=jnp.float32,
                precision=lax.Precision.HIGHEST)
    h = h + hr_ref[...]
    h = jnp.maximum(h, 0.0)
    z = jnp.dot(h, wo_ref[...], preferred_element_type=jnp.float32,
                precision=lax.Precision.HIGHEST) + bo_ref[...]
    out_ref[...] = jax.nn.sigmoid(z)


@functools.lru_cache(maxsize=None)
def _make_dense(N, D, C, BN):
    grid = (N // BN,)

    def _z(i):
        return jnp.zeros_like(i)

    return pl.pallas_call(
        _dense_body,
        grid=grid,
        in_specs=[
            pl.BlockSpec((_NC, BN, D), lambda i: (_z(i), i, _z(i))),
            pl.BlockSpec((BN, _NW), lambda i: (i, _z(i))),
            pl.BlockSpec((BN, D), lambda i: (i, _z(i))),
            pl.BlockSpec((D, D), lambda i: (_z(i), _z(i))),
            pl.BlockSpec((D, C), lambda i: (_z(i), _z(i))),
            pl.BlockSpec((1, C), lambda i: (_z(i), _z(i))),
        ],
        out_specs=pl.BlockSpec((BN, C), lambda i: (i, _z(i))),
        out_shape=jax.ShapeDtypeStruct((N, C), jnp.float32),
    )


def kernel(x, edge_index, W_l, b_l, W_r, W_out, b_out):
    N, D = x.shape
    E = edge_index.shape[1]
    C = W_out.shape[0]

    agg_fn, NP, NPAD = _make_agg(N, E, D)

    ei = edge_index.astype(jnp.int32)
    EW = E // _NW
    src = ei[0].reshape(_NW, EW)
    dst = ei[1].reshape(_NW, EW)
    pad = ((0, 0), (0, NP - EW))
    src = jnp.pad(src, pad).reshape(_NW, NP // _K, _K)
    dst = jnp.pad(dst, pad, constant_values=N).reshape(_NW, NP // _K, _K)
    zeros = jnp.zeros((NPAD, D), jnp.float32)

    hr = _make_dense1(N, D, 2000)(x, W_r.T, b_l.reshape(1, D))
    acc_flat, deg_flat = agg_fn(src, dst, x, zeros)
    acc = acc_flat.reshape(_NC, N, D)
    deg = deg_flat.reshape(_NW, N).T

    out = _make_dense(N, D, C, 2000)(
        acc, deg, hr, W_l.T, W_out.T, b_out.reshape(1, C))
    return out


# speedup vs baseline: 1.6069x; 1.0169x over previous
import functools

import jax
import jax.numpy as jnp
from jax import lax
from jax.experimental import pallas as pl
from jax.experimental.pallas import tpu as pltpu
from jax.experimental.pallas import tpu_sc as plsc

_NC = 2
_NS = 16
_NW = _NC * _NS
_K = 80
_NBUF = 2


@functools.lru_cache(maxsize=None)
def _make_agg(N, E, D):
    EW = -(-E // _NW)
    n_chunks = ((EW + _K - 1) // _K + 1) // 2 * 2
    NP = n_chunks * _K
    NPAD = (N + _NS * 8 - 1) // (_NS * 8) * (_NS * 8)
    zstripe = NPAD // _NS
    stripe = (N // _NS) // 8 * 8
    tail = N - stripe * _NS

    mesh = plsc.VectorSubcoreMesh(core_axis_name="c", subcore_axis_name="s")

    @functools.partial(
        pl.kernel,
        out_type=[
            jax.ShapeDtypeStruct((_NC * N, D), jnp.float32),
            jax.ShapeDtypeStruct((_NW * N,), jnp.float32),
        ],
        mesh=mesh,
        scratch_types=[
            pltpu.VMEM((2, _K), jnp.int32),
            pltpu.VMEM((2, _K), jnp.int32),
            pltpu.VMEM((2, _K, D), jnp.float32),
            pltpu.VMEM((NPAD,), jnp.float32),
            pltpu.VMEM_SHARED((NPAD, D), jnp.float32),
            [pltpu.SemaphoreType.DMA] * 2,
            [pltpu.SemaphoreType.DMA] * 2,
            pltpu.SemaphoreType.DMA,
        ],
        compiler_params=pltpu.CompilerParams(needs_layout_passes=False),
    )
    def agg(src_h, dst_h, x_h, acc_out, deg_out,
            src_v, dst_v, rows_v, deg_v, acc_sh, gsems, isems, ssem):
        i32 = jnp.int32
        cid = lax.axis_index("c")
        sid = lax.axis_index("s")
        wid = sid * i32(_NC) + cid

        def zrow(r, carry):
            for j in range(D // 16):
                rows_v[i32(0), r, pl.ds(i32(j * 16), 16)] = (
                    jnp.zeros((16,), jnp.float32))
            return carry
        lax.fori_loop(i32(0), i32(_K), zrow, i32(0))

        zoff = sid * i32(zstripe)
        nfull, rem = divmod(zstripe, _K)
        for k in range(nfull):
            pltpu.make_async_copy(
                rows_v.at[i32(0)],
                acc_sh.at[pl.ds(zoff + i32(k * _K), _K)], ssem).start()
        if rem:
            pltpu.make_async_copy(
                rows_v.at[i32(0), pl.ds(i32(0), rem)],
                acc_sh.at[pl.ds(zoff + i32(nfull * _K), rem)], ssem).start()
        for k in range(nfull):
            pltpu.make_async_copy(
                rows_v.at[i32(0)],
                acc_sh.at[pl.ds(zoff + i32(k * _K), _K)], ssem).wait()
        if rem:
            pltpu.make_async_copy(
                rows_v.at[i32(0), pl.ds(i32(0), rem)],
                acc_sh.at[pl.ds(zoff + i32(nfull * _K), rem)], ssem).wait()

        def zbody(i, carry):
            deg_v[pl.ds(i * i32(16), 16)] = jnp.zeros((16,), jnp.float32)
            return carry
        lax.fori_loop(i32(0), i32(NPAD // 16), zbody, i32(0))

        plsc.subcore_barrier()

        ones = jnp.ones((16,), jnp.float32)

        def iload(c, p):
            for ref, hbm in ((src_v, src_h), (dst_v, dst_h)):
                pltpu.make_async_copy(hbm.at[wid, c], ref.at[i32(p)],
                                      isems[p]).start()

        def iload_wait(p):
            for ref, hbm in ((src_v, src_h), (dst_v, dst_h)):
                pltpu.make_async_copy(hbm.at[wid, i32(0)], ref.at[i32(p)],
                                      isems[p]).wait()

        def gather(p):
            pltpu.make_async_copy(x_h.at[src_v.at[i32(p)]],
                                  rows_v.at[i32(p)], gsems[p]).start()

        def gather_wait(p):
            pltpu.make_async_copy(x_h.at[src_v.at[i32(0)]],
                                  rows_v.at[i32(p)], gsems[p]).wait()

        iload(i32(0), 0)
        iload_wait(0)
        gather(0)
        iload(i32(1), 1)

        def half(c, p):
            @pl.when(c < i32(n_chunks - 1))
            def _next_gather():
                iload_wait(1 - p)
                gather(1 - p)
            gather_wait(p)
            scat = pltpu.make_async_copy(rows_v.at[i32(p)],
                                         acc_sh.at[dst_v.at[i32(p)]], ssem)
            scat.start(add=True)
            for j in range(_K // 16):
                dvec = dst_v[i32(p), pl.ds(i32(j * 16), 16)]
                plsc.addupdate_scatter(deg_v, [dvec], ones)
            scat.wait()

            @pl.when(c < i32(n_chunks - 2))
            def _next_iload():
                iload(c + i32(2), p)

        def step(s, carry):
            half(s * i32(2), 0)
            half(s * i32(2) + i32(1), 1)
            return carry
        lax.fori_loop(i32(0), i32(n_chunks // 2), step, i32(0))

        plsc.subcore_barrier()

        soff = sid * i32(stripe)
        pltpu.sync_copy(acc_sh.at[pl.ds(soff, stripe)],
                        acc_out.at[pl.ds(cid * i32(N) + soff, stripe)])
        if tail:
            @pl.when(sid == _NS - 1)
            def _out_tail():
                pltpu.sync_copy(
                    acc_sh.at[pl.ds(_NS * stripe, tail)],
                    acc_out.at[pl.ds(cid * i32(N) + i32(_NS * stripe), tail)])
        pltpu.sync_copy(deg_v.at[pl.ds(i32(0), N)],
                        deg_out.at[pl.ds(wid * i32(N), N)])

    return agg, NP, NPAD


def _dense1_body(x_ref, wr_ref, bl_ref, hr_ref):
    hr_ref[...] = jnp.dot(
        x_ref[...], wr_ref[...], preferred_element_type=jnp.float32,
        precision=lax.Precision.HIGHEST) + bl_ref[...]


@functools.lru_cache(maxsize=None)
def _make_dense1(N, D, BN):
    def _z(i):
        return jnp.zeros_like(i)

    return pl.pallas_call(
        _dense1_body,
        grid=(N // BN,),
        in_specs=[
            pl.BlockSpec((BN, D), lambda i: (i, _z(i))),
            pl.BlockSpec((D, D), lambda i: (_z(i), _z(i))),
            pl.BlockSpec((1, D), lambda i: (_z(i), _z(i))),
        ],
        out_specs=pl.BlockSpec((BN, D), lambda i: (i, _z(i))),
        out_shape=jax.ShapeDtypeStruct((N, D), jnp.float32),
    )


def _dense_body(acc_ref, deg_ref, hr_ref, wl_ref, wo_ref, bo_ref, out_ref):
    agg_sum = acc_ref[0] + acc_ref[1]
    deg = jnp.sum(deg_ref[...], axis=1, keepdims=True)
    agg = agg_sum * (1.0 / jnp.maximum(deg, 1.0))
    h = jnp.dot(agg, wl_ref[...], preferred_element_type=jnp.float32,
                precision=lax.Precision.HIGHEST)
    h = h + hr_ref[...]
    h = jnp.maximum(h, 0.0)
    z = jnp.dot(h, wo_ref[...], preferred_element_type=jnp.float32,
                precision=lax.Precision.HIGHEST) + bo_ref[...]
    out_ref[...] = jax.nn.sigmoid(z)


@functools.lru_cache(maxsize=None)
def _make_dense(N, D, C, BN):
    grid = (N // BN,)

    def _z(i):
        return jnp.zeros_like(i)

    return pl.pallas_call(
        _dense_body,
        grid=grid,
        in_specs=[
            pl.BlockSpec((_NC, BN, D), lambda i: (_z(i), i, _z(i))),
            pl.BlockSpec((BN, _NW), lambda i: (i, _z(i))),
            pl.BlockSpec((BN, D), lambda i: (i, _z(i))),
            pl.BlockSpec((D, D), lambda i: (_z(i), _z(i))),
            pl.BlockSpec((D, C), lambda i: (_z(i), _z(i))),
            pl.BlockSpec((1, C), lambda i: (_z(i), _z(i))),
        ],
        out_specs=pl.BlockSpec((BN, C), lambda i: (i, _z(i))),
        out_shape=jax.ShapeDtypeStruct((N, C), jnp.float32),
    )


def kernel(x, edge_index, W_l, b_l, W_r, W_out, b_out):
    N, D = x.shape
    E = edge_index.shape[1]
    C = W_out.shape[0]

    agg_fn, NP, NPAD = _make_agg(N, E, D)

    ei = edge_index.astype(jnp.int32)
    EW = E // _NW
    src = ei[0].reshape(_NW, EW)
    dst = ei[1].reshape(_NW, EW)
    pad = ((0, 0), (0, NP - EW))
    src = jnp.pad(src, pad).reshape(_NW, NP // _K, _K)
    dst = jnp.pad(dst, pad, constant_values=N).reshape(_NW, NP // _K, _K)

    hr = _make_dense1(N, D, 2000)(x, W_r.T, b_l.reshape(1, D))
    acc_flat, deg_flat = agg_fn(src, dst, x)
    acc = acc_flat.reshape(_NC, N, D)
    deg = deg_flat.reshape(_NW, N).T

    out = _make_dense(N, D, C, 2000)(
        acc, deg, hr, W_l.T, W_out.T, b_out.reshape(1, C))
    return out


# speedup vs baseline: 1.6538x; 1.0292x over previous
import functools

import jax
import jax.numpy as jnp
from jax import lax
from jax.experimental import pallas as pl
from jax.experimental.pallas import tpu as pltpu
from jax.experimental.pallas import tpu_sc as plsc

_NC = 2
_NS = 16
_NW = _NC * _NS
_K = 80
_NBUF = 2


@functools.lru_cache(maxsize=None)
def _make_agg(N, E, D):
    EW = -(-E // _NW)
    n_chunks = ((EW + _K - 1) // _K + 1) // 2 * 2
    NP = n_chunks * _K
    NPAD = (N + _NS * 8 - 1) // (_NS * 8) * (_NS * 8)
    zstripe = NPAD // _NS
    stripe = (N // _NS) // 8 * 8
    tail = N - stripe * _NS

    mesh = plsc.VectorSubcoreMesh(core_axis_name="c", subcore_axis_name="s")

    @functools.partial(
        pl.kernel,
        out_type=[
            jax.ShapeDtypeStruct((_NC * N, D), jnp.float32),
            jax.ShapeDtypeStruct((_NW * N,), jnp.float32),
        ],
        mesh=mesh,
        scratch_types=[
            pltpu.VMEM((2, _K), jnp.int32),
            pltpu.VMEM((2, _K), jnp.int32),
            pltpu.VMEM((2, _K, D), jnp.float32),
            pltpu.VMEM((NPAD,), jnp.float32),
            pltpu.VMEM_SHARED((NPAD, D), jnp.float32),
            [pltpu.SemaphoreType.DMA] * 2,
            [pltpu.SemaphoreType.DMA] * 2,
            pltpu.SemaphoreType.DMA,
        ],
        compiler_params=pltpu.CompilerParams(needs_layout_passes=False),
    )
    def agg(src_h, dst_h, x_h, acc_out, deg_out,
            src_v, dst_v, rows_v, deg_v, acc_sh, gsems, isems, ssem):
        i32 = jnp.int32
        cid = lax.axis_index("c")
        sid = lax.axis_index("s")
        wid = sid * i32(_NC) + cid

        def zrow(r, carry):
            for j in range(D // 16):
                rows_v[i32(0), r, pl.ds(i32(j * 16), 16)] = (
                    jnp.zeros((16,), jnp.float32))
            return carry
        lax.fori_loop(i32(0), i32(_K), zrow, i32(0))

        zoff = sid * i32(zstripe)
        nfull, rem = divmod(zstripe, _K)
        for k in range(nfull):
            pltpu.make_async_copy(
                rows_v.at[i32(0)],
                acc_sh.at[pl.ds(zoff + i32(k * _K), _K)], ssem).start()
        if rem:
            pltpu.make_async_copy(
                rows_v.at[i32(0), pl.ds(i32(0), rem)],
                acc_sh.at[pl.ds(zoff + i32(nfull * _K), rem)], ssem).start()
        for k in range(nfull):
            pltpu.make_async_copy(
                rows_v.at[i32(0)],
                acc_sh.at[pl.ds(zoff + i32(k * _K), _K)], ssem).wait()
        if rem:
            pltpu.make_async_copy(
                rows_v.at[i32(0), pl.ds(i32(0), rem)],
                acc_sh.at[pl.ds(zoff + i32(nfull * _K), rem)], ssem).wait()

        def zbody(i, carry):
            deg_v[pl.ds(i * i32(16), 16)] = jnp.zeros((16,), jnp.float32)
            return carry
        lax.fori_loop(i32(0), i32(NPAD // 16), zbody, i32(0))

        plsc.subcore_barrier()

        ones = jnp.ones((16,), jnp.float32)

        def iload(c, p):
            for ref, hbm in ((src_v, src_h), (dst_v, dst_h)):
                pltpu.make_async_copy(hbm.at[wid, c], ref.at[i32(p)],
                                      isems[p]).start()

        def iload_wait(p):
            for ref, hbm in ((src_v, src_h), (dst_v, dst_h)):
                pltpu.make_async_copy(hbm.at[wid, i32(0)], ref.at[i32(p)],
                                      isems[p]).wait()

        def gather(p):
            pltpu.make_async_copy(x_h.at[src_v.at[i32(p)]],
                                  rows_v.at[i32(p)], gsems[p]).start()

        def gather_wait(p):
            pltpu.make_async_copy(x_h.at[src_v.at[i32(0)]],
                                  rows_v.at[i32(p)], gsems[p]).wait()

        iload(i32(0), 0)
        iload_wait(0)
        gather(0)
        iload(i32(1), 1)

        def half(c, p):
            @pl.when(c < i32(n_chunks - 1))
            def _next_gather():
                iload_wait(1 - p)
                gather(1 - p)
            gather_wait(p)
            scat = pltpu.make_async_copy(rows_v.at[i32(p)],
                                         acc_sh.at[dst_v.at[i32(p)]], ssem)
            scat.start(add=True)
            for j in range(_K // 16):
                dvec = dst_v[i32(p), pl.ds(i32(j * 16), 16)]
                plsc.addupdate_scatter(deg_v, [dvec], ones)
            scat.wait()

            @pl.when(c < i32(n_chunks - 2))
            def _next_iload():
                iload(c + i32(2), p)

        def step(s, carry):
            half(s * i32(2), 0)
            half(s * i32(2) + i32(1), 1)
            return carry
        lax.fori_loop(i32(0), i32(n_chunks // 2), step, i32(0))

        plsc.subcore_barrier()

        soff = sid * i32(stripe)
        pltpu.sync_copy(acc_sh.at[pl.ds(soff, stripe)],
                        acc_out.at[pl.ds(cid * i32(N) + soff, stripe)])
        if tail:
            @pl.when(sid == _NS - 1)
            def _out_tail():
                pltpu.sync_copy(
                    acc_sh.at[pl.ds(_NS * stripe, tail)],
                    acc_out.at[pl.ds(cid * i32(N) + i32(_NS * stripe), tail)])
        pltpu.sync_copy(deg_v.at[pl.ds(i32(0), N)],
                        deg_out.at[pl.ds(wid * i32(N), N)])

    return agg, NP, NPAD


def _dense1_body(x_ref, wr_ref, bl_ref, hr_ref):
    hr_ref[...] = jnp.dot(
        x_ref[...], wr_ref[...], preferred_element_type=jnp.float32) + bl_ref[...]


@functools.lru_cache(maxsize=None)
def _make_dense1(N, D, BN):
    def _z(i):
        return jnp.zeros_like(i)

    return pl.pallas_call(
        _dense1_body,
        grid=(N // BN,),
        in_specs=[
            pl.BlockSpec((BN, D), lambda i: (i, _z(i))),
            pl.BlockSpec((D, D), lambda i: (_z(i), _z(i))),
            pl.BlockSpec((1, D), lambda i: (_z(i), _z(i))),
        ],
        out_specs=pl.BlockSpec((BN, D), lambda i: (i, _z(i))),
        out_shape=jax.ShapeDtypeStruct((N, D), jnp.float32),
    )


def _dense_body(acc_ref, deg_ref, hr_ref, wl_ref, wo_ref, bo_ref, out_ref):
    agg_sum = acc_ref[0] + acc_ref[1]
    deg = jnp.sum(deg_ref[...], axis=1, keepdims=True)
    agg = agg_sum * (1.0 / jnp.maximum(deg, 1.0))
    h = jnp.dot(agg, wl_ref[...], preferred_element_type=jnp.float32)
    h = h + hr_ref[...]
    h = jnp.maximum(h, 0.0)
    z = jnp.dot(h, wo_ref[...], preferred_element_type=jnp.float32) + bo_ref[...]
    out_ref[...] = jax.nn.sigmoid(z)


@functools.lru_cache(maxsize=None)
def _make_dense(N, D, C, BN):
    grid = (N // BN,)

    def _z(i):
        return jnp.zeros_like(i)

    return pl.pallas_call(
        _dense_body,
        grid=grid,
        in_specs=[
            pl.BlockSpec((_NC, BN, D), lambda i: (_z(i), i, _z(i))),
            pl.BlockSpec((BN, _NW), lambda i: (i, _z(i))),
            pl.BlockSpec((BN, D), lambda i: (i, _z(i))),
            pl.BlockSpec((D, D), lambda i: (_z(i), _z(i))),
            pl.BlockSpec((D, C), lambda i: (_z(i), _z(i))),
            pl.BlockSpec((1, C), lambda i: (_z(i), _z(i))),
        ],
        out_specs=pl.BlockSpec((BN, C), lambda i: (i, _z(i))),
        out_shape=jax.ShapeDtypeStruct((N, C), jnp.float32),
    )


def kernel(x, edge_index, W_l, b_l, W_r, W_out, b_out):
    N, D = x.shape
    E = edge_index.shape[1]
    C = W_out.shape[0]

    agg_fn, NP, NPAD = _make_agg(N, E, D)

    ei = edge_index.astype(jnp.int32)
    EW = E // _NW
    src = ei[0].reshape(_NW, EW)
    dst = ei[1].reshape(_NW, EW)
    pad = ((0, 0), (0, NP - EW))
    src = jnp.pad(src, pad).reshape(_NW, NP // _K, _K)
    dst = jnp.pad(dst, pad, constant_values=N).reshape(_NW, NP // _K, _K)

    hr = _make_dense1(N, D, 2000)(x, W_r.T, b_l.reshape(1, D))
    acc_flat, deg_flat = agg_fn(src, dst, x)
    acc = acc_flat.reshape(_NC, N, D)
    deg = deg_flat.reshape(_NW, N).T

    out = _make_dense(N, D, C, 2000)(
        acc, deg, hr, W_l.T, W_out.T, b_out.reshape(1, C))
    return out


# speedup vs baseline: 1.6604x; 1.0040x over previous
import functools

import jax
import jax.numpy as jnp
from jax import lax
from jax.experimental import pallas as pl
from jax.experimental.pallas import tpu as pltpu
from jax.experimental.pallas import tpu_sc as plsc

_NC = 2
_NS = 16
_NW = _NC * _NS
_K = 80
_NBUF = 2


@functools.lru_cache(maxsize=None)
def _make_agg(N, E, D):
    EW = -(-E // _NW)
    n_chunks = ((EW + _K - 1) // _K + 1) // 2 * 2
    NP = n_chunks * _K
    NPAD = (N + _NS * 8 - 1) // (_NS * 8) * (_NS * 8)
    zstripe = NPAD // _NS
    stripe = (N // _NS) // 8 * 8
    tail = N - stripe * _NS

    mesh = plsc.VectorSubcoreMesh(core_axis_name="c", subcore_axis_name="s")

    @functools.partial(
        pl.kernel,
        out_type=[
            jax.ShapeDtypeStruct((_NC * N, D), jnp.float32),
            jax.ShapeDtypeStruct((_NW * N,), jnp.float32),
        ],
        mesh=mesh,
        scratch_types=[
            pltpu.VMEM((2, _K), jnp.int32),
            pltpu.VMEM((2, _K), jnp.int32),
            pltpu.VMEM((2, _K, D), jnp.float32),
            pltpu.VMEM((NPAD,), jnp.float32),
            pltpu.VMEM_SHARED((NPAD, D), jnp.float32),
            [pltpu.SemaphoreType.DMA] * 2,
            [pltpu.SemaphoreType.DMA] * 2,
            pltpu.SemaphoreType.DMA,
        ],
        compiler_params=pltpu.CompilerParams(needs_layout_passes=False),
    )
    def agg(src_h, dst_h, x_h, acc_out, deg_out,
            src_v, dst_v, rows_v, deg_v, acc_sh, gsems, isems, ssem):
        i32 = jnp.int32
        cid = lax.axis_index("c")
        sid = lax.axis_index("s")
        wid = sid * i32(_NC) + cid

        def zrow(r, carry):
            for j in range(D // 16):
                rows_v[i32(0), r, pl.ds(i32(j * 16), 16)] = (
                    jnp.zeros((16,), jnp.float32))
            return carry
        lax.fori_loop(i32(0), i32(_K), zrow, i32(0))

        zoff = sid * i32(zstripe)
        nfull, rem = divmod(zstripe, _K)
        for k in range(nfull):
            pltpu.make_async_copy(
                rows_v.at[i32(0)],
                acc_sh.at[pl.ds(zoff + i32(k * _K), _K)], ssem).start()
        if rem:
            pltpu.make_async_copy(
                rows_v.at[i32(0), pl.ds(i32(0), rem)],
                acc_sh.at[pl.ds(zoff + i32(nfull * _K), rem)], ssem).start()
        for k in range(nfull):
            pltpu.make_async_copy(
                rows_v.at[i32(0)],
                acc_sh.at[pl.ds(zoff + i32(k * _K), _K)], ssem).wait()
        if rem:
            pltpu.make_async_copy(
                rows_v.at[i32(0), pl.ds(i32(0), rem)],
                acc_sh.at[pl.ds(zoff + i32(nfull * _K), rem)], ssem).wait()

        def zbody(i, carry):
            deg_v[pl.ds(i * i32(16), 16)] = jnp.zeros((16,), jnp.float32)
            return carry
        lax.fori_loop(i32(0), i32(NPAD // 16), zbody, i32(0))

        plsc.subcore_barrier()

        ones = jnp.ones((16,), jnp.float32)

        def iload(c, p):
            for ref, hbm in ((src_v, src_h), (dst_v, dst_h)):
                pltpu.make_async_copy(hbm.at[wid, c], ref.at[i32(p)],
                                      isems[p]).start()

        def iload_wait(p):
            for ref, hbm in ((src_v, src_h), (dst_v, dst_h)):
                pltpu.make_async_copy(hbm.at[wid, i32(0)], ref.at[i32(p)],
                                      isems[p]).wait()

        def gather(p):
            pltpu.make_async_copy(x_h.at[src_v.at[i32(p)]],
                                  rows_v.at[i32(p)], gsems[p]).start()

        def gather_wait(p):
            pltpu.make_async_copy(x_h.at[src_v.at[i32(0)]],
                                  rows_v.at[i32(p)], gsems[p]).wait()

        iload(i32(0), 0)
        iload_wait(0)
        gather(0)
        iload(i32(1), 1)

        def half(c, p):
            @pl.when(c < i32(n_chunks - 1))
            def _next_gather():
                iload_wait(1 - p)
                gather(1 - p)
            gather_wait(p)
            scat = pltpu.make_async_copy(rows_v.at[i32(p)],
                                         acc_sh.at[dst_v.at[i32(p)]], ssem)
            scat.start(add=True)
            for j in range(_K // 16):
                dvec = dst_v[i32(p), pl.ds(i32(j * 16), 16)]
                plsc.addupdate_scatter(deg_v, [dvec], ones)
            scat.wait()

            @pl.when(c < i32(n_chunks - 2))
            def _next_iload():
                iload(c + i32(2), p)

        def step(s, carry):
            half(s * i32(2), 0)
            half(s * i32(2) + i32(1), 1)
            return carry
        lax.fori_loop(i32(0), i32(n_chunks // 2), step, i32(0))

        plsc.subcore_barrier()

        soff = sid * i32(stripe)
        pltpu.sync_copy(acc_sh.at[pl.ds(soff, stripe)],
                        acc_out.at[pl.ds(cid * i32(N) + soff, stripe)])
        if tail:
            @pl.when(sid == _NS - 1)
            def _out_tail():
                pltpu.sync_copy(
                    acc_sh.at[pl.ds(_NS * stripe, tail)],
                    acc_out.at[pl.ds(cid * i32(N) + i32(_NS * stripe), tail)])
        pltpu.sync_copy(deg_v.at[pl.ds(i32(0), N)],
                        deg_out.at[pl.ds(wid * i32(N), N)])

    return agg, NP, NPAD


def _dense1_body(x_ref, wr_ref, bl_ref, hr_ref):
    hr_ref[...] = jnp.dot(
        x_ref[...], wr_ref[...], preferred_element_type=jnp.float32) + bl_ref[...]


@functools.lru_cache(maxsize=None)
def _make_dense1(N, D, BN):
    def _z(i):
        return jnp.zeros_like(i)

    return pl.pallas_call(
        _dense1_body,
        grid=(N // BN,),
        in_specs=[
            pl.BlockSpec((BN, D), lambda i: (i, _z(i))),
            pl.BlockSpec((D, D), lambda i: (_z(i), _z(i))),
            pl.BlockSpec((1, D), lambda i: (_z(i), _z(i))),
        ],
        out_specs=pl.BlockSpec((BN, D), lambda i: (i, _z(i))),
        out_shape=jax.ShapeDtypeStruct((N, D), jnp.float32),
    )


def _dense_body(acc_ref, deg_ref, hr_ref, wl_ref, wo_ref, bo_ref, out_ref):
    agg_sum = acc_ref[0] + acc_ref[1]
    deg = jnp.sum(deg_ref[...], axis=1, keepdims=True)
    agg = agg_sum * (1.0 / jnp.maximum(deg, 1.0))
    h = jnp.dot(agg, wl_ref[...], preferred_element_type=jnp.float32)
    h = h + hr_ref[...]
    h = jnp.maximum(h, 0.0)
    z = jnp.dot(h, wo_ref[...], preferred_element_type=jnp.float32) + bo_ref[...]
    out_ref[...] = jax.nn.sigmoid(z)


@functools.lru_cache(maxsize=None)
def _make_dense(N, D, C, BN):
    grid = (N // BN,)

    def _z(i):
        return jnp.zeros_like(i)

    return pl.pallas_call(
        _dense_body,
        grid=grid,
        in_specs=[
            pl.BlockSpec((_NC, BN, D), lambda i: (_z(i), i, _z(i))),
            pl.BlockSpec((BN, _NW), lambda i: (i, _z(i))),
            pl.BlockSpec((BN, D), lambda i: (i, _z(i))),
            pl.BlockSpec((D, D), lambda i: (_z(i), _z(i))),
            pl.BlockSpec((D, C), lambda i: (_z(i), _z(i))),
            pl.BlockSpec((1, C), lambda i: (_z(i), _z(i))),
        ],
        out_specs=pl.BlockSpec((BN, C), lambda i: (i, _z(i))),
        out_shape=jax.ShapeDtypeStruct((N, C), jnp.float32),
    )


def kernel(x, edge_index, W_l, b_l, W_r, W_out, b_out):
    N, D = x.shape
    E = edge_index.shape[1]
    C = W_out.shape[0]

    agg_fn, NP, NPAD = _make_agg(N, E, D)

    ei = edge_index.astype(jnp.int32)
    EW = E // _NW
    src = ei[0].reshape(_NW, EW)
    dst = ei[1].reshape(_NW, EW)
    pad = ((0, 0), (0, NP - EW))
    src = jnp.pad(src, pad).reshape(_NW, NP // _K, _K)
    dst = jnp.pad(dst, pad, constant_values=N).reshape(_NW, NP // _K, _K)

    hr = _make_dense1(N, D, 5000)(x, W_r.T, b_l.reshape(1, D))
    acc_flat, deg_flat = agg_fn(src, dst, x)
    acc = acc_flat.reshape(_NC, N, D)
    deg = deg_flat.reshape(_NW, N).T

    out = _make_dense(N, D, C, 5000)(
        acc, deg, hr, W_l.T, W_out.T, b_out.reshape(1, C))
    return out
